# baseline (speedup 1.0000x reference)
import functools

import jax
import jax.numpy as jnp
from jax import lax
from jax.experimental import pallas as pl
from jax.experimental.pallas import tpu as pltpu
from jax.experimental.pallas import tpu_sc as plsc

_G = 512
_LANES = 16
_NC = 2
_NS = 16


def _segment_max_sc(x, batch):
    n, d = x.shape
    nw = _NC * _NS
    rows_per_w = n // nw
    blk = 125
    nblk = rows_per_w // blk
    acc_words = _G * d
    ids_pad = rows_per_w + 8 - rows_per_w % 8

    batch2 = jnp.pad(batch.reshape(nw, rows_per_w),
                     ((0, 0), (0, ids_pad - rows_per_w)))

    mesh = plsc.VectorSubcoreMesh(
        core_axis_name="c", subcore_axis_name="s",
        num_cores=_NC, num_subcores=_NS)

    @functools.partial(
        pl.kernel,
        mesh=mesh,
        compiler_params=pltpu.CompilerParams(use_tc_tiling_on_sc=False),
        out_type=(
            jax.ShapeDtypeStruct((nw, acc_words), jnp.float32),
            jax.ShapeDtypeStruct((nw, 8), jnp.int32),
        ),
        scratch_types=[
            pltpu.VMEM((acc_words,), jnp.float32),
            pltpu.VMEM((ids_pad + _LANES,), jnp.int32),
            pltpu.VMEM((2, blk, d), jnp.float32),
            pltpu.VMEM((_LANES,), jnp.int32),
            pltpu.SemaphoreType.DMA,
            pltpu.SemaphoreType.DMA,
        ],
    )
    def seg_max(x_hbm, b_hbm, out_hbm, rng_hbm, acc, bids, xbuf, rngv,
                sem0, sem1):
        wid = lax.axis_index("s") * _NC + lax.axis_index("c")
        base = wid * rows_per_w

        pltpu.sync_copy(b_hbm.at[wid], bids.at[pl.ds(0, ids_pad)])

        pltpu.async_copy(x_hbm.at[pl.ds(base, blk)], xbuf.at[0], sem0)
        pltpu.async_copy(x_hbm.at[pl.ds(base + blk, blk)], xbuf.at[1], sem1)

        nvec = d // _LANES
        ngrp = blk // _LANES
        tail = blk - ngrp * _LANES

        lo = bids[pl.ds(0, _LANES)][0]
        hi = bids[pl.ds(rows_per_w - 1, _LANES)][0]

        neg = jnp.full((_LANES,), -jnp.inf, jnp.float32)

        def init_body(s, _):
            off = s * d
            for j in range(nvec):
                acc[pl.ds(off + j * _LANES, _LANES)] = neg
            return 0

        lax.fori_loop(lo, hi + 1, init_body, 0)

        iot = lax.iota(jnp.int32, _LANES)
        rngv[...] = jnp.where(iot == 0, jnp.full((_LANES,), lo),
                              jnp.full((_LANES,), hi))
        pltpu.sync_copy(rngv.at[pl.ds(0, 8)], rng_hbm.at[wid])

        def rows_16(row0, slot, idsv, nrows):
            def loads(k):
                return [xbuf[slot, row0 + k, pl.ds(j * _LANES, _LANES)]
                        for j in range(nvec)]

            xv = loads(0)
            for k in range(nrows):
                nxt = loads(k + 1) if k + 1 < nrows else None
                seg = idsv[k]
                off = seg * d
                av = [acc[pl.ds(off + j * _LANES, _LANES)]
                      for j in range(nvec)]
                for j in range(nvec):
                    acc[pl.ds(off + j * _LANES, _LANES)] = (
                        jnp.maximum(av[j], xv[j]))
                xv = nxt

        def block_body(bi, c):
            slot = lax.rem(bi, 2)
            @pl.when(slot == 0)
            def _():
                pltpu.make_async_copy(
                    x_hbm.at[pl.ds(base, blk)], xbuf.at[0], sem0).wait()

            @pl.when(slot == 1)
            def _():
                pltpu.make_async_copy(
                    x_hbm.at[pl.ds(base, blk)], xbuf.at[1], sem1).wait()

            def run_group(row0, idsv, nrows):
                s0 = idsv[0]
                slast = idsv[nrows - 1]

                def fast():
                    xs = [[xbuf[slot, row0 + k, pl.ds(j * _LANES, _LANES)]
                           for j in range(nvec)] for k in range(nrows)]
                    off = s0 * d
                    av = [acc[pl.ds(off + j * _LANES, _LANES)]
                          for j in range(nvec)]
                    for j in range(nvec):
                        col = [xs[k][j] for k in range(nrows)] + [av[j]]
                        while len(col) > 1:
                            col = [jnp.maximum(col[i], col[i + 1])
                                   if i + 1 < len(col) else col[i]
                                   for i in range(0, len(col), 2)]
                        acc[pl.ds(off + j * _LANES, _LANES)] = col[0]

                def slow():
                    rows_16(row0, slot, idsv, nrows)

                lax.cond(s0 == slast, fast, slow)

            def group_body(g, _):
                row0 = g * _LANES
                run_group(row0, bids[pl.ds(bi * blk + row0, _LANES)], _LANES)
                return 0

            lax.fori_loop(0, ngrp, group_body, 0)
            idsv = bids[pl.ds(bi * blk + ngrp * _LANES, _LANES)]
            rows_16(ngrp * _LANES, slot, idsv, tail)

            @pl.when((slot == 0) & (bi + 2 < nblk))
            def _():
                pltpu.async_copy(
                    x_hbm.at[pl.ds(base + (bi + 2) * blk, blk)],
                    xbuf.at[0], sem0)

            @pl.when((slot == 1) & (bi + 2 < nblk))
            def _():
                pltpu.async_copy(
                    x_hbm.at[pl.ds(base + (bi + 2) * blk, blk)],
                    xbuf.at[1], sem1)

            return 0

        lax.fori_loop(0, nblk, block_body, 0)

        cwords = 32 * d
        for ci in range(_G // 32):
            @pl.when((ci * 32 <= hi) & (ci * 32 + 31 >= lo))
            def _(ci=ci):
                pltpu.sync_copy(acc.at[pl.ds(ci * cwords, cwords)],
                                out_hbm.at[wid, pl.ds(ci * cwords, cwords)])

    return seg_max(x, batch2)


def _combine_and_matmul_tc(accs, rng, w_pad):
    nw = accs.shape[0]

    def body(a_ref, rng_ref, w_ref, o_ref):
        a = a_ref[...]
        r = rng_ref[...]
        lo = r[:, 0].reshape(nw, 1, 1)
        hi = r[:, 1].reshape(nw, 1, 1)
        gi = lax.broadcasted_iota(jnp.int32, (nw, a.shape[1], 1), 1)
        am = jnp.where((gi >= lo) & (gi <= hi), a, -jnp.inf)
        hg = jnp.max(am, axis=0)
        o_ref[...] = jnp.dot(hg, w_ref[...],
                             preferred_element_type=jnp.float32)

    return pl.pallas_call(
        body,
        out_shape=jax.ShapeDtypeStruct((_G, 128), jnp.float32),
    )(accs, rng, w_pad)


def kernel(x, batch, W):
    n, d = x.shape
    n_classes = W.shape[0]
    batch = batch.astype(jnp.int32)
    accs, rng = _segment_max_sc(x, batch)
    accs = accs.reshape(_NC * _NS, _G, d)
    w_pad = jnp.zeros((d, 128), jnp.float32).at[:, :n_classes].set(W.T)
    logits = _combine_and_matmul_tc(accs, rng, w_pad)
    return logits[:, :n_classes]


# speedup vs baseline: 1.0626x; 1.0626x over previous
import functools

import jax
import jax.numpy as jnp
from jax import lax
from jax.experimental import pallas as pl
from jax.experimental.pallas import tpu as pltpu
from jax.experimental.pallas import tpu_sc as plsc

_G = 512
_LANES = 16
_NC = 2
_NS = 16


def _segment_max_sc(x, batch):
    n, d = x.shape
    nw = _NC * _NS
    rows_per_w = n // nw
    blk = 125
    nblk = rows_per_w // blk
    acc_words = _G * d
    ids_pad = rows_per_w + 8 - rows_per_w % 8

    batch2 = jnp.pad(batch.reshape(nw, rows_per_w),
                     ((0, 0), (0, ids_pad - rows_per_w)))

    mesh = plsc.VectorSubcoreMesh(
        core_axis_name="c", subcore_axis_name="s",
        num_cores=_NC, num_subcores=_NS)

    @functools.partial(
        pl.kernel,
        mesh=mesh,
        compiler_params=pltpu.CompilerParams(use_tc_tiling_on_sc=False),
        out_type=(
            jax.ShapeDtypeStruct((nw, acc_words), jnp.float32),
            jax.ShapeDtypeStruct((nw, 8), jnp.int32),
        ),
        scratch_types=[
            pltpu.VMEM((acc_words,), jnp.float32),
            pltpu.VMEM((ids_pad + _LANES,), jnp.int32),
            pltpu.VMEM((2, blk, d), jnp.float32),
            pltpu.VMEM((_LANES,), jnp.int32),
            pltpu.SemaphoreType.DMA,
            pltpu.SemaphoreType.DMA,
        ],
    )
    def seg_max(x_hbm, b_hbm, out_hbm, rng_hbm, acc, bids, xbuf, rngv,
                sem0, sem1):
        wid = lax.axis_index("s") * _NC + lax.axis_index("c")
        base = wid * rows_per_w

        pltpu.sync_copy(b_hbm.at[wid], bids.at[pl.ds(0, ids_pad)])

        pltpu.async_copy(x_hbm.at[pl.ds(base, blk)], xbuf.at[0], sem0)
        pltpu.async_copy(x_hbm.at[pl.ds(base + blk, blk)], xbuf.at[1], sem1)

        nvec = d // _LANES
        ngrp = blk // _LANES
        tail = blk - ngrp * _LANES

        lo = bids[pl.ds(0, _LANES)][0]
        hi = bids[pl.ds(rows_per_w - 1, _LANES)][0]

        neg = jnp.full((_LANES,), -jnp.inf, jnp.float32)

        def init_body(s, _):
            off = s * d
            for j in range(nvec):
                acc[pl.ds(off + j * _LANES, _LANES)] = neg
            return 0

        lax.fori_loop(lo, hi + 1, init_body, 0)

        iot = lax.iota(jnp.int32, _LANES)
        rngv[...] = jnp.where(iot == 0, jnp.full((_LANES,), lo),
                              jnp.full((_LANES,), hi))
        pltpu.sync_copy(rngv.at[pl.ds(0, 8)], rng_hbm.at[wid])

        def rows_16(row0, slot, idsv, nrows):
            def loads(k):
                return [xbuf[slot, row0 + k, pl.ds(j * _LANES, _LANES)]
                        for j in range(nvec)]

            xv = loads(0)
            for k in range(nrows):
                nxt = loads(k + 1) if k + 1 < nrows else None
                seg = idsv[k]
                off = seg * d
                av = [acc[pl.ds(off + j * _LANES, _LANES)]
                      for j in range(nvec)]
                for j in range(nvec):
                    acc[pl.ds(off + j * _LANES, _LANES)] = (
                        jnp.maximum(av[j], xv[j]))
                xv = nxt

        def block_body(bi, c):
            slot = lax.rem(bi, 2)
            @pl.when(slot == 0)
            def _():
                pltpu.make_async_copy(
                    x_hbm.at[pl.ds(base, blk)], xbuf.at[0], sem0).wait()

            @pl.when(slot == 1)
            def _():
                pltpu.make_async_copy(
                    x_hbm.at[pl.ds(base, blk)], xbuf.at[1], sem1).wait()

            def run_group(row0, idsv, nrows):
                s0 = idsv[0]
                slast = idsv[nrows - 1]

                def fast():
                    xs = [[xbuf[slot, row0 + k, pl.ds(j * _LANES, _LANES)]
                           for j in range(nvec)] for k in range(nrows)]
                    off = s0 * d
                    av = [acc[pl.ds(off + j * _LANES, _LANES)]
                          for j in range(nvec)]
                    for j in range(nvec):
                        col = [xs[k][j] for k in range(nrows)] + [av[j]]
                        while len(col) > 1:
                            col = [jnp.maximum(col[i], col[i + 1])
                                   if i + 1 < len(col) else col[i]
                                   for i in range(0, len(col), 2)]
                        acc[pl.ds(off + j * _LANES, _LANES)] = col[0]

                def slow():
                    rows_16(row0, slot, idsv, nrows)

                lax.cond(s0 == slast, fast, slow)

            def group_body(g, _):
                row0 = g * _LANES
                run_group(row0, bids[pl.ds(bi * blk + row0, _LANES)], _LANES)
                return 0

            lax.fori_loop(0, ngrp, group_body, 0)
            idsv = bids[pl.ds(bi * blk + ngrp * _LANES, _LANES)]
            rows_16(ngrp * _LANES, slot, idsv, tail)

            @pl.when((slot == 0) & (bi + 2 < nblk))
            def _():
                pltpu.async_copy(
                    x_hbm.at[pl.ds(base + (bi + 2) * blk, blk)],
                    xbuf.at[0], sem0)

            @pl.when((slot == 1) & (bi + 2 < nblk))
            def _():
                pltpu.async_copy(
                    x_hbm.at[pl.ds(base + (bi + 2) * blk, blk)],
                    xbuf.at[1], sem1)

            return 0

        lax.fori_loop(0, nblk, block_body, 0)

        cwords = 32 * d
        for ci in range(_G // 32):
            @pl.when((ci * 32 <= hi) & (ci * 32 + 31 >= lo))
            def _(ci=ci):
                pltpu.sync_copy(acc.at[pl.ds(ci * cwords, cwords)],
                                out_hbm.at[wid, pl.ds(ci * cwords, cwords)])

    return seg_max(x, batch2)


def _combine_and_matmul_tc(accs, rng, w_pad):
    nw = accs.shape[0]

    def body(a_ref, rng_ref, w_ref, o_ref):
        a = a_ref[...]
        r = rng_ref[...]
        lo = r[:, 0].reshape(nw, 1, 1)
        hi = r[:, 1].reshape(nw, 1, 1)
        gi = lax.broadcasted_iota(jnp.int32, a.shape, 1)
        am = jnp.where((gi >= lo) & (gi <= hi), a, -jnp.inf)
        hg = jnp.max(am, axis=0)
        o_ref[...] = jnp.dot(hg, w_ref[...],
                             preferred_element_type=jnp.float32)

    return pl.pallas_call(
        body,
        out_shape=jax.ShapeDtypeStruct((_G, 128), jnp.float32),
    )(accs, rng, w_pad)


def kernel(x, batch, W):
    n, d = x.shape
    n_classes = W.shape[0]
    batch = batch.astype(jnp.int32)
    accs, rng = _segment_max_sc(x, batch)
    accs = accs.reshape(_NC * _NS, _G, d)
    w_pad = jnp.zeros((d, 128), jnp.float32).at[:, :n_classes].set(W.T)
    logits = _combine_and_matmul_tc(accs, rng, w_pad)
    return logits[:, :n_classes]


# speedup vs baseline: 1.0998x; 1.0349x over previous
import functools

import jax
import jax.numpy as jnp
from jax import lax
from jax.experimental import pallas as pl
from jax.experimental.pallas import tpu as pltpu
from jax.experimental.pallas import tpu_sc as plsc

_G = 512
_LANES = 16
_NC = 2
_NS = 16


def _segment_max_sc(x, batch):
    n, d = x.shape
    nw = _NC * _NS
    rows_per_w = n // nw
    blk = 125
    nblk = rows_per_w // blk
    acc_words = _G * d
    ids_pad = rows_per_w + 8 - rows_per_w % 8

    batch2 = jnp.pad(batch.reshape(nw, rows_per_w),
                     ((0, 0), (0, ids_pad - rows_per_w)))

    mesh = plsc.VectorSubcoreMesh(
        core_axis_name="c", subcore_axis_name="s",
        num_cores=_NC, num_subcores=_NS)

    @functools.partial(
        pl.kernel,
        mesh=mesh,
        compiler_params=pltpu.CompilerParams(use_tc_tiling_on_sc=False),
        out_type=(
            jax.ShapeDtypeStruct((nw, acc_words), jnp.float32),
            jax.ShapeDtypeStruct((nw, 8), jnp.int32),
        ),
        scratch_types=[
            pltpu.VMEM((acc_words,), jnp.float32),
            pltpu.VMEM((ids_pad + _LANES,), jnp.int32),
            pltpu.VMEM((3, blk, d), jnp.float32),
            pltpu.VMEM((_LANES,), jnp.int32),
            pltpu.SemaphoreType.DMA,
            pltpu.SemaphoreType.DMA,
            pltpu.SemaphoreType.DMA,
        ],
    )
    def seg_max(x_hbm, b_hbm, out_hbm, rng_hbm, acc, bids, xbuf, rngv,
                sem0, sem1, sem2):
        wid = lax.axis_index("s") * _NC + lax.axis_index("c")
        base = wid * rows_per_w

        pltpu.sync_copy(b_hbm.at[wid], bids.at[pl.ds(0, ids_pad)])

        pltpu.async_copy(x_hbm.at[pl.ds(base, blk)], xbuf.at[0], sem0)
        pltpu.async_copy(x_hbm.at[pl.ds(base + blk, blk)], xbuf.at[1], sem1)
        pltpu.async_copy(x_hbm.at[pl.ds(base + 2 * blk, blk)], xbuf.at[2],
                         sem2)

        nvec = d // _LANES
        ngrp = blk // _LANES
        tail = blk - ngrp * _LANES

        lo = bids[pl.ds(0, _LANES)][0]
        hi = bids[pl.ds(rows_per_w - 1, _LANES)][0]

        neg = jnp.full((_LANES,), -jnp.inf, jnp.float32)

        def init_body(s, _):
            off = s * d
            for j in range(nvec):
                acc[pl.ds(off + j * _LANES, _LANES)] = neg
            return 0

        lax.fori_loop(lo, hi + 1, init_body, 0)

        iot = lax.iota(jnp.int32, _LANES)
        rngv[...] = jnp.where(iot == 0, jnp.full((_LANES,), lo),
                              jnp.full((_LANES,), hi))
        pltpu.sync_copy(rngv.at[pl.ds(0, 8)], rng_hbm.at[wid])

        def rows_16(row0, slot, idsv, nrows):
            def loads(k):
                return [xbuf[slot, row0 + k, pl.ds(j * _LANES, _LANES)]
                        for j in range(nvec)]

            xv = loads(0)
            for k in range(nrows):
                nxt = loads(k + 1) if k + 1 < nrows else None
                seg = idsv[k]
                off = seg * d
                av = [acc[pl.ds(off + j * _LANES, _LANES)]
                      for j in range(nvec)]
                for j in range(nvec):
                    acc[pl.ds(off + j * _LANES, _LANES)] = (
                        jnp.maximum(av[j], xv[j]))
                xv = nxt

        sems = (sem0, sem1, sem2)

        def block_body(bi, c):
            slot = lax.rem(bi, 3)
            for s in range(3):
                @pl.when(slot == s)
                def _(s=s):
                    pltpu.make_async_copy(
                        x_hbm.at[pl.ds(base, blk)], xbuf.at[s],
                        sems[s]).wait()

            def run_group(row0, idsv, nrows):
                s0 = idsv[0]
                slast = idsv[nrows - 1]

                def fast():
                    xs = [[xbuf[slot, row0 + k, pl.ds(j * _LANES, _LANES)]
                           for j in range(nvec)] for k in range(nrows)]
                    off = s0 * d
                    av = [acc[pl.ds(off + j * _LANES, _LANES)]
                          for j in range(nvec)]
                    for j in range(nvec):
                        col = [xs[k][j] for k in range(nrows)] + [av[j]]
                        while len(col) > 1:
                            col = [jnp.maximum(col[i], col[i + 1])
                                   if i + 1 < len(col) else col[i]
                                   for i in range(0, len(col), 2)]
                        acc[pl.ds(off + j * _LANES, _LANES)] = col[0]

                def slow():
                    rows_16(row0, slot, idsv, nrows)

                lax.cond(s0 == slast, fast, slow)

            def group_body(g, _):
                row0 = g * _LANES
                run_group(row0, bids[pl.ds(bi * blk + row0, _LANES)], _LANES)
                return 0

            lax.fori_loop(0, ngrp, group_body, 0)
            idsv = bids[pl.ds(bi * blk + ngrp * _LANES, _LANES)]
            rows_16(ngrp * _LANES, slot, idsv, tail)

            for s in range(3):
                @pl.when((slot == s) & (bi + 3 < nblk))
                def _(s=s):
                    pltpu.async_copy(
                        x_hbm.at[pl.ds(base + (bi + 3) * blk, blk)],
                        xbuf.at[s], sems[s])

            return 0

        lax.fori_loop(0, nblk, block_body, 0)

        cwords = 32 * d
        for ci in range(_G // 32):
            @pl.when((ci * 32 <= hi) & (ci * 32 + 31 >= lo))
            def _(ci=ci):
                pltpu.sync_copy(acc.at[pl.ds(ci * cwords, cwords)],
                                out_hbm.at[wid, pl.ds(ci * cwords, cwords)])

    return seg_max(x, batch2)


def _combine_and_matmul_tc(accs, rng, w_pad):
    nw = accs.shape[0]

    def body(a_ref, rng_ref, w_ref, o_ref):
        a = a_ref[...]
        r = rng_ref[...]
        lo = r[:, 0].reshape(nw, 1, 1)
        hi = r[:, 1].reshape(nw, 1, 1)
        gi = lax.broadcasted_iota(jnp.int32, a.shape, 1)
        am = jnp.where((gi >= lo) & (gi <= hi), a, -jnp.inf)
        hg = jnp.max(am, axis=0)
        o_ref[...] = jnp.dot(hg, w_ref[...],
                             preferred_element_type=jnp.float32)

    return pl.pallas_call(
        body,
        out_shape=jax.ShapeDtypeStruct((_G, 128), jnp.float32),
    )(accs, rng, w_pad)


def kernel(x, batch, W):
    n, d = x.shape
    n_classes = W.shape[0]
    batch = batch.astype(jnp.int32)
    accs, rng = _segment_max_sc(x, batch)
    accs = accs.reshape(_NC * _NS, _G, d)
    w_pad = jnp.zeros((d, 128), jnp.float32).at[:, :n_classes].set(W.T)
    logits = _combine_and_matmul_tc(accs, rng, w_pad)
    return logits[:, :n_classes]
